# baseline (device time: 43094 ns/iter reference)
import jax
import jax.numpy as jnp
from jax import lax
from jax.experimental import pallas as pl
from jax.experimental.pallas import tpu as pltpu

N_DEV = 4
M_PER = 64
M = N_DEV * M_PER
HALF = M // 2
D = 512
H_PER = 1024
BF = jnp.bfloat16


def kernel(x, Win0, Wout0, Win1, Wout1, Win2, Wout2):
    def body(x_ref, win0_ref, wout0_ref, win1_ref, wout1_ref,
             win2_ref, wout2_ref, out_ref,
             xfull, hbuf, zbuf, recvbuf,
             w1buf, w2buf, wsems, send_sems, recv_sems):
        p = lax.axis_index("i")
        a = p ^ 1
        b = 3 - p
        mychunk = p * M_PER
        abase = a * M_PER
        gbase = (p // 2) * HALF
        obase = (1 - p // 2) * HALF

        wdmas = []
        for i, (src, dst) in enumerate((
                (win0_ref, w1buf.at[0]), (wout0_ref, w2buf.at[0]),
                (win1_ref, w1buf.at[1]), (wout1_ref, w2buf.at[1]),
                (win2_ref, w1buf.at[2]), (wout2_ref, w2buf.at[2]))):
            dma = pltpu.make_async_copy(src, dst, wsems.at[i])
            dma.start()
            wdmas.append(dma)

        barrier_sem = pltpu.get_barrier_semaphore()
        for nbr in (a, b):
            pl.semaphore_signal(
                barrier_sem, inc=1,
                device_id=(nbr,), device_id_type=pl.DeviceIdType.MESH,
            )
        pl.semaphore_wait(barrier_sem, 2)

        def exchange(idx, partner, src, dst):
            rdma = pltpu.make_async_remote_copy(
                src_ref=src, dst_ref=dst,
                send_sem=send_sems.at[idx], recv_sem=recv_sems.at[idx],
                device_id=(partner,), device_id_type=pl.DeviceIdType.MESH,
            )
            rdma.start()
            return rdma

        xfull[pl.ds(mychunk, M_PER), :] = x_ref[:, :].astype(BF)
        e0 = exchange(0, a,
                      xfull.at[pl.ds(mychunk, M_PER)],
                      xfull.at[pl.ds(mychunk, M_PER)])
        e0.wait()
        e1 = exchange(1, b,
                      xfull.at[pl.ds(gbase, HALF)],
                      xfull.at[pl.ds(gbase, HALF)])
        wdmas[0].wait()
        W1 = w1buf[0, :, :].astype(BF)
        hg = jnp.dot(xfull[pl.ds(gbase, HALF), :], W1,
                     preferred_element_type=jnp.float32)
        hbuf[pl.ds(gbase, HALF), :] = jnp.maximum(hg, 0.0).astype(BF)
        e1.wait()
        ho = jnp.dot(xfull[pl.ds(obase, HALF), :], W1,
                     preferred_element_type=jnp.float32)
        hbuf[pl.ds(obase, HALF), :] = jnp.maximum(ho, 0.0).astype(BF)

        def halfmm(src, r, w):
            return jnp.dot(src[pl.ds(r * HALF, HALF), :], w,
                           preferred_element_type=jnp.float32)

        def allreduce_fused(lyr, sem0):
            wdmas[2 * lyr + 1].wait()
            W2 = w2buf[lyr, :, :].astype(BF)
            zbuf[pl.ds(0, HALF), :] = halfmm(hbuf, 0, W2).astype(BF)
            eB0 = exchange(sem0, b,
                           zbuf.at[pl.ds(0, HALF)],
                           recvbuf.at[0, pl.ds(0, HALF)])
            zbuf[pl.ds(HALF, HALF), :] = halfmm(hbuf, 1, W2).astype(BF)
            eB1 = exchange(sem0 + 1, b,
                           zbuf.at[pl.ds(HALF, HALF)],
                           recvbuf.at[0, pl.ds(HALF, HALF)])
            eB0.wait()
            zbuf[pl.ds(0, HALF), :] = (
                zbuf[pl.ds(0, HALF), :] + recvbuf[0, pl.ds(0, HALF), :]
            )
            eA0 = exchange(sem0 + 2, a,
                           zbuf.at[pl.ds(0, HALF)],
                           recvbuf.at[1, pl.ds(0, HALF)])
            eB1.wait()
            zbuf[pl.ds(HALF, HALF), :] = (
                zbuf[pl.ds(HALF, HALF), :] + recvbuf[0, pl.ds(HALF, HALF), :]
            )
            eA1 = exchange(sem0 + 3, a,
                           zbuf.at[pl.ds(HALF, HALF)],
                           recvbuf.at[1, pl.ds(HALF, HALF)])
            wdmas[2 * lyr + 2].wait()
            W1n = w1buf[lyr + 1, :, :].astype(BF)
            eA0.wait()
            xfull[pl.ds(0, HALF), :] = (
                zbuf[pl.ds(0, HALF), :] + recvbuf[1, pl.ds(0, HALF), :]
            )
            h0 = halfmm(xfull, 0, W1n)
            hbuf[pl.ds(0, HALF), :] = jnp.maximum(h0, 0.0).astype(BF)
            eA1.wait()
            xfull[pl.ds(HALF, HALF), :] = (
                zbuf[pl.ds(HALF, HALF), :] + recvbuf[1, pl.ds(HALF, HALF), :]
            )
            h1 = halfmm(xfull, 1, W1n)
            hbuf[pl.ds(HALF, HALF), :] = jnp.maximum(h1, 0.0).astype(BF)

        allreduce_fused(0, 2)
        allreduce_fused(1, 6)

        wdmas[5].wait()
        W2 = w2buf[2, :, :].astype(BF)
        zo = jnp.dot(hbuf[pl.ds(obase, HALF), :], W2,
                     preferred_element_type=jnp.float32)
        zbuf[pl.ds(obase, HALF), :] = zo.astype(BF)
        eRB = exchange(10, b,
                       zbuf.at[pl.ds(obase, HALF)],
                       recvbuf.at[0, pl.ds(obase, HALF)])
        zg = jnp.dot(hbuf[pl.ds(gbase, HALF), :], W2,
                     preferred_element_type=jnp.float32)
        zbuf[pl.ds(gbase, HALF), :] = zg.astype(BF)
        eRB.wait()
        zbuf[pl.ds(gbase, HALF), :] = (
            zbuf[pl.ds(gbase, HALF), :] + recvbuf[0, pl.ds(gbase, HALF), :]
        )
        eRA = exchange(11, a,
                       zbuf.at[pl.ds(abase, M_PER)],
                       recvbuf.at[1, pl.ds(abase, M_PER)])
        eRA.wait()
        out_ref[:, :] = (
            zbuf[pl.ds(mychunk, M_PER), :]
            + recvbuf[1, pl.ds(mychunk, M_PER), :]
        )

    return pl.pallas_call(
        body,
        out_shape=jax.ShapeDtypeStruct((M_PER, D), BF),
        in_specs=[pl.BlockSpec(memory_space=pltpu.VMEM)]
        + [pl.BlockSpec(memory_space=pl.ANY)] * 6,
        out_specs=pl.BlockSpec(memory_space=pltpu.VMEM),
        scratch_shapes=[
            pltpu.VMEM((M, D), BF),
            pltpu.VMEM((M, H_PER), BF),
            pltpu.VMEM((M, D), BF),
            pltpu.VMEM((2, M, D), BF),
            pltpu.VMEM((3, D, H_PER), jnp.float32),
            pltpu.VMEM((3, H_PER, D), jnp.float32),
            pltpu.SemaphoreType.DMA((6,)),
            pltpu.SemaphoreType.DMA((12,)),
            pltpu.SemaphoreType.DMA((12,)),
        ],
        compiler_params=pltpu.CompilerParams(collective_id=0),
    )(x, Win0, Wout0, Win1, Wout1, Win2, Wout2)


# device time: 43079 ns/iter; 1.0003x vs baseline; 1.0003x over previous
import jax
import jax.numpy as jnp
from jax import lax
from jax.experimental import pallas as pl
from jax.experimental.pallas import tpu as pltpu

N_DEV = 4
M_PER = 64
M = N_DEV * M_PER
HALF = M // 2
D = 512
H_PER = 1024
BF = jnp.bfloat16


def kernel(x, Win0, Wout0, Win1, Wout1, Win2, Wout2):
    def body(x_ref, win0_ref, wout0_ref, win1_ref, wout1_ref,
             win2_ref, wout2_ref, out_ref,
             xfull, hbuf, zbuf, recvbuf,
             w1buf, w2buf, wsems, send_sems, recv_sems):
        p = lax.axis_index("i")
        a = p ^ 1
        b = 3 - p
        mychunk = p * M_PER
        abase = a * M_PER
        gbase = (p // 2) * HALF
        obase = (1 - p // 2) * HALF

        wdmas = []
        for i, (src, dst) in enumerate((
                (win0_ref, w1buf.at[0]), (wout0_ref, w2buf.at[0]),
                (win1_ref, w1buf.at[1]), (wout1_ref, w2buf.at[1]),
                (win2_ref, w1buf.at[2]), (wout2_ref, w2buf.at[2]))):
            dma = pltpu.make_async_copy(src, dst, wsems.at[i])
            dma.start()
            wdmas.append(dma)

        barrier_sem = pltpu.get_barrier_semaphore()
        for nbr in (a, b):
            pl.semaphore_signal(
                barrier_sem, inc=1,
                device_id=(nbr,), device_id_type=pl.DeviceIdType.MESH,
            )
        pl.semaphore_wait(barrier_sem, 2)

        def exchange(idx, partner, src, dst):
            rdma = pltpu.make_async_remote_copy(
                src_ref=src, dst_ref=dst,
                send_sem=send_sems.at[idx], recv_sem=recv_sems.at[idx],
                device_id=(partner,), device_id_type=pl.DeviceIdType.MESH,
            )
            rdma.start()
            return rdma

        xfull[pl.ds(mychunk, M_PER), :] = x_ref[:, :].astype(BF)
        e0 = exchange(0, a,
                      xfull.at[pl.ds(mychunk, M_PER)],
                      xfull.at[pl.ds(mychunk, M_PER)])
        e0.wait()
        e1 = exchange(1, b,
                      xfull.at[pl.ds(gbase, HALF)],
                      xfull.at[pl.ds(gbase, HALF)])
        wdmas[0].wait()
        W1 = w1buf[0, :, :].astype(BF)
        hg = jnp.dot(xfull[pl.ds(gbase, HALF), :], W1,
                     preferred_element_type=jnp.float32)
        hbuf[pl.ds(gbase, HALF), :] = jnp.maximum(hg, 0.0).astype(BF)
        e1.wait()
        ho = jnp.dot(xfull[pl.ds(obase, HALF), :], W1,
                     preferred_element_type=jnp.float32)
        hbuf[pl.ds(obase, HALF), :] = jnp.maximum(ho, 0.0).astype(BF)

        def halfmm(src, r, w):
            return jnp.dot(src[pl.ds(r * HALF, HALF), :], w,
                           preferred_element_type=jnp.float32)

        def allreduce_fused(lyr, sem0):
            wdmas[2 * lyr + 1].wait()
            W2 = w2buf[lyr, :, :].astype(BF)
            zbuf[pl.ds(0, HALF), :] = halfmm(hbuf, 0, W2).astype(BF)
            eB0 = exchange(sem0, b,
                           zbuf.at[pl.ds(0, HALF)],
                           recvbuf.at[0, pl.ds(0, HALF)])
            zbuf[pl.ds(HALF, HALF), :] = halfmm(hbuf, 1, W2).astype(BF)
            eB1 = exchange(sem0 + 1, b,
                           zbuf.at[pl.ds(HALF, HALF)],
                           recvbuf.at[0, pl.ds(HALF, HALF)])
            eB0.wait()
            zbuf[pl.ds(0, HALF), :] = (
                zbuf[pl.ds(0, HALF), :] + recvbuf[0, pl.ds(0, HALF), :]
            )
            eA0 = exchange(sem0 + 2, a,
                           zbuf.at[pl.ds(0, HALF)],
                           recvbuf.at[1, pl.ds(0, HALF)])
            eB1.wait()
            zbuf[pl.ds(HALF, HALF), :] = (
                zbuf[pl.ds(HALF, HALF), :] + recvbuf[0, pl.ds(HALF, HALF), :]
            )
            eA1 = exchange(sem0 + 3, a,
                           zbuf.at[pl.ds(HALF, HALF)],
                           recvbuf.at[1, pl.ds(HALF, HALF)])
            wdmas[2 * lyr + 2].wait()
            W1n = w1buf[lyr + 1, :, :].astype(BF)
            eA0.wait()
            xfull[pl.ds(0, HALF), :] = (
                zbuf[pl.ds(0, HALF), :] + recvbuf[1, pl.ds(0, HALF), :]
            )
            h0 = halfmm(xfull, 0, W1n)
            hbuf[pl.ds(0, HALF), :] = jnp.maximum(h0, 0.0).astype(BF)
            eA1.wait()
            xfull[pl.ds(HALF, HALF), :] = (
                zbuf[pl.ds(HALF, HALF), :] + recvbuf[1, pl.ds(HALF, HALF), :]
            )
            h1 = halfmm(xfull, 1, W1n)
            hbuf[pl.ds(HALF, HALF), :] = jnp.maximum(h1, 0.0).astype(BF)

        allreduce_fused(0, 2)
        allreduce_fused(1, 6)

        wdmas[5].wait()
        W2 = w2buf[2, :, :].astype(BF)
        zo = jnp.dot(hbuf[pl.ds(obase, HALF), :], W2,
                     preferred_element_type=jnp.float32)
        zbuf[pl.ds(obase, HALF), :] = zo.astype(BF)
        eRB = exchange(10, b,
                       zbuf.at[pl.ds(obase, HALF)],
                       recvbuf.at[0, pl.ds(obase, HALF)])
        zg = jnp.dot(hbuf[pl.ds(gbase, HALF), :], W2,
                     preferred_element_type=jnp.float32)
        zbuf[pl.ds(gbase, HALF), :] = zg.astype(BF)
        eRB.wait()
        zbuf[pl.ds(gbase, HALF), :] = (
            zbuf[pl.ds(gbase, HALF), :] + recvbuf[0, pl.ds(gbase, HALF), :]
        )
        eRA = exchange(11, a,
                       zbuf.at[pl.ds(abase, M_PER)],
                       recvbuf.at[1, pl.ds(abase, M_PER)])
        eRA.wait()
        out_ref[:, :] = (
            zbuf[pl.ds(mychunk, M_PER), :]
            + recvbuf[1, pl.ds(mychunk, M_PER), :]
        )

    return pl.pallas_call(
        body,
        out_shape=jax.ShapeDtypeStruct((M_PER, D), BF),
        in_specs=[pl.BlockSpec(memory_space=pltpu.VMEM)]
        + [pl.BlockSpec(memory_space=pltpu.MemorySpace.HBM)] * 6,
        out_specs=pl.BlockSpec(memory_space=pltpu.VMEM),
        scratch_shapes=[
            pltpu.VMEM((M, D), BF),
            pltpu.VMEM((M, H_PER), BF),
            pltpu.VMEM((M, D), BF),
            pltpu.VMEM((2, M, D), BF),
            pltpu.VMEM((3, D, H_PER), jnp.float32),
            pltpu.VMEM((3, H_PER, D), jnp.float32),
            pltpu.SemaphoreType.DMA((6,)),
            pltpu.SemaphoreType.DMA((12,)),
            pltpu.SemaphoreType.DMA((12,)),
        ],
        compiler_params=pltpu.CompilerParams(collective_id=0),
    )(x, Win0, Wout0, Win1, Wout1, Win2, Wout2)


# device time: 31939 ns/iter; 1.3493x vs baseline; 1.3488x over previous
import jax
import jax.numpy as jnp
from jax import lax
from jax.experimental import pallas as pl
from jax.experimental.pallas import tpu as pltpu

N_DEV = 4
M_PER = 64
M = N_DEV * M_PER
HALF = M // 2
D = 512
H_PER = 1024
BF = jnp.bfloat16


def kernel(x, Win0, Wout0, Win1, Wout1, Win2, Wout2):
    def body(x_ref, win0_ref, wout0_ref, win1_ref, wout1_ref,
             win2_ref, wout2_ref, out_ref,
             xfull, hbuf, zbuf, recvbuf,
             xf32, outstage, w1buf, w2buf, wsems, send_sems, recv_sems):
        p = lax.axis_index("i")
        a = p ^ 1
        b = 3 - p
        mychunk = p * M_PER
        abase = a * M_PER
        gbase = (p // 2) * HALF
        obase = (1 - p // 2) * HALF

        xdma = pltpu.make_async_copy(x_ref, xf32, wsems.at[6])
        xdma.start()
        wdmas = []
        for i, (src, dst) in enumerate((
                (win0_ref, w1buf.at[0]), (wout0_ref, w2buf.at[0]),
                (win1_ref, w1buf.at[1]), (wout1_ref, w2buf.at[1]),
                (win2_ref, w1buf.at[2]), (wout2_ref, w2buf.at[2]))):
            dma = pltpu.make_async_copy(src, dst, wsems.at[i])
            dma.start()
            wdmas.append(dma)

        barrier_sem = pltpu.get_barrier_semaphore()
        for nbr in (a, b):
            pl.semaphore_signal(
                barrier_sem, inc=1,
                device_id=(nbr,), device_id_type=pl.DeviceIdType.MESH,
            )
        pl.semaphore_wait(barrier_sem, 2)

        def exchange(idx, partner, src, dst):
            rdma = pltpu.make_async_remote_copy(
                src_ref=src, dst_ref=dst,
                send_sem=send_sems.at[idx], recv_sem=recv_sems.at[idx],
                device_id=(partner,), device_id_type=pl.DeviceIdType.MESH,
            )
            rdma.start()
            return rdma

        xdma.wait()
        xfull[pl.ds(mychunk, M_PER), :] = xf32[:, :].astype(BF)
        e0 = exchange(0, a,
                      xfull.at[pl.ds(mychunk, M_PER)],
                      xfull.at[pl.ds(mychunk, M_PER)])
        e0.wait()
        e1 = exchange(1, b,
                      xfull.at[pl.ds(gbase, HALF)],
                      xfull.at[pl.ds(gbase, HALF)])
        wdmas[0].wait()
        W1 = w1buf[0, :, :].astype(BF)
        hg = jnp.dot(xfull[pl.ds(gbase, HALF), :], W1,
                     preferred_element_type=jnp.float32)
        hbuf[pl.ds(gbase, HALF), :] = jnp.maximum(hg, 0.0).astype(BF)
        e1.wait()
        ho = jnp.dot(xfull[pl.ds(obase, HALF), :], W1,
                     preferred_element_type=jnp.float32)
        hbuf[pl.ds(obase, HALF), :] = jnp.maximum(ho, 0.0).astype(BF)

        def halfmm(src, r, w):
            return jnp.dot(src[pl.ds(r * HALF, HALF), :], w,
                           preferred_element_type=jnp.float32)

        def allreduce_fused(lyr, sem0):
            wdmas[2 * lyr + 1].wait()
            W2 = w2buf[lyr, :, :].astype(BF)
            zbuf[pl.ds(0, HALF), :] = halfmm(hbuf, 0, W2).astype(BF)
            eB0 = exchange(sem0, b,
                           zbuf.at[pl.ds(0, HALF)],
                           recvbuf.at[0, pl.ds(0, HALF)])
            zbuf[pl.ds(HALF, HALF), :] = halfmm(hbuf, 1, W2).astype(BF)
            eB1 = exchange(sem0 + 1, b,
                           zbuf.at[pl.ds(HALF, HALF)],
                           recvbuf.at[0, pl.ds(HALF, HALF)])
            eB0.wait()
            zbuf[pl.ds(0, HALF), :] = (
                zbuf[pl.ds(0, HALF), :] + recvbuf[0, pl.ds(0, HALF), :]
            )
            eA0 = exchange(sem0 + 2, a,
                           zbuf.at[pl.ds(0, HALF)],
                           recvbuf.at[1, pl.ds(0, HALF)])
            eB1.wait()
            zbuf[pl.ds(HALF, HALF), :] = (
                zbuf[pl.ds(HALF, HALF), :] + recvbuf[0, pl.ds(HALF, HALF), :]
            )
            eA1 = exchange(sem0 + 3, a,
                           zbuf.at[pl.ds(HALF, HALF)],
                           recvbuf.at[1, pl.ds(HALF, HALF)])
            wdmas[2 * lyr + 2].wait()
            W1n = w1buf[lyr + 1, :, :].astype(BF)
            eA0.wait()
            xfull[pl.ds(0, HALF), :] = (
                zbuf[pl.ds(0, HALF), :] + recvbuf[1, pl.ds(0, HALF), :]
            )
            h0 = halfmm(xfull, 0, W1n)
            hbuf[pl.ds(0, HALF), :] = jnp.maximum(h0, 0.0).astype(BF)
            eA1.wait()
            xfull[pl.ds(HALF, HALF), :] = (
                zbuf[pl.ds(HALF, HALF), :] + recvbuf[1, pl.ds(HALF, HALF), :]
            )
            h1 = halfmm(xfull, 1, W1n)
            hbuf[pl.ds(HALF, HALF), :] = jnp.maximum(h1, 0.0).astype(BF)

        allreduce_fused(0, 2)
        allreduce_fused(1, 6)

        wdmas[5].wait()
        W2 = w2buf[2, :, :].astype(BF)
        zo = jnp.dot(hbuf[pl.ds(obase, HALF), :], W2,
                     preferred_element_type=jnp.float32)
        zbuf[pl.ds(obase, HALF), :] = zo.astype(BF)
        eRB = exchange(10, b,
                       zbuf.at[pl.ds(obase, HALF)],
                       recvbuf.at[0, pl.ds(obase, HALF)])
        zg = jnp.dot(hbuf[pl.ds(gbase, HALF), :], W2,
                     preferred_element_type=jnp.float32)
        zbuf[pl.ds(gbase, HALF), :] = zg.astype(BF)
        eRB.wait()
        zbuf[pl.ds(gbase, HALF), :] = (
            zbuf[pl.ds(gbase, HALF), :] + recvbuf[0, pl.ds(gbase, HALF), :]
        )
        eRA = exchange(11, a,
                       zbuf.at[pl.ds(abase, M_PER)],
                       recvbuf.at[1, pl.ds(abase, M_PER)])
        eRA.wait()
        outstage[:, :] = (
            zbuf[pl.ds(mychunk, M_PER), :]
            + recvbuf[1, pl.ds(mychunk, M_PER), :]
        )
        odma = pltpu.make_async_copy(outstage, out_ref, wsems.at[7])
        odma.start()
        odma.wait()

    hbm = pltpu.MemorySpace.HBM
    args = tuple(
        pltpu.with_memory_space_constraint(v, hbm)
        for v in (x, Win0, Wout0, Win1, Wout1, Win2, Wout2)
    )
    return pl.pallas_call(
        body,
        out_shape=jax.ShapeDtypeStruct((M_PER, D), BF),
        in_specs=[pl.BlockSpec(memory_space=hbm)] * 7,
        out_specs=pl.BlockSpec(memory_space=hbm),
        scratch_shapes=[
            pltpu.VMEM((M, D), BF),
            pltpu.VMEM((M, H_PER), BF),
            pltpu.VMEM((M, D), BF),
            pltpu.VMEM((2, M, D), BF),
            pltpu.VMEM((M_PER, D), jnp.float32),
            pltpu.VMEM((M_PER, D), BF),
            pltpu.VMEM((3, D, H_PER), jnp.float32),
            pltpu.VMEM((3, H_PER, D), jnp.float32),
            pltpu.SemaphoreType.DMA((8,)),
            pltpu.SemaphoreType.DMA((12,)),
            pltpu.SemaphoreType.DMA((12,)),
        ],
        compiler_params=pltpu.CompilerParams(collective_id=0),
    )(*args)


# device time: 28774 ns/iter; 1.4977x vs baseline; 1.1100x over previous
import jax
import jax.numpy as jnp
from jax import lax
from jax.experimental import pallas as pl
from jax.experimental.pallas import tpu as pltpu

N_DEV = 4
M_PER = 64
M = N_DEV * M_PER
HALF = M // 2
D = 512
DH = D // 2
H_PER = 1024
BF = jnp.bfloat16


def kernel(x, Win0, Wout0, Win1, Wout1, Win2, Wout2):
    def body(x_ref, win0_ref, wout0_ref, win1_ref, wout1_ref,
             win2_ref, wout2_ref, out_ref,
             xfull, hbuf, zbuf, recvbuf,
             xf32, outstage, w1buf, w2buf, wsems, send_sems, recv_sems):
        p = lax.axis_index("i")
        a = p ^ 1
        b = 3 - p
        mychunk = p * M_PER
        abase = a * M_PER
        bbase = b * M_PER
        dbase = (p ^ 2) * M_PER
        gbase = (p // 2) * HALF
        obase = (1 - p // 2) * HALF

        xdma = pltpu.make_async_copy(x_ref, xf32, wsems.at[6])
        xdma.start()
        wdmas = []
        for i, (src, dst) in enumerate((
                (win0_ref, w1buf.at[0]), (wout0_ref, w2buf.at[0]),
                (win1_ref, w1buf.at[1]), (wout1_ref, w2buf.at[1]),
                (win2_ref, w1buf.at[2]), (wout2_ref, w2buf.at[2]))):
            dma = pltpu.make_async_copy(src, dst, wsems.at[i])
            dma.start()
            wdmas.append(dma)

        barrier_sem = pltpu.get_barrier_semaphore()
        for nbr in (a, b):
            pl.semaphore_signal(
                barrier_sem, inc=1,
                device_id=(nbr,), device_id_type=pl.DeviceIdType.MESH,
            )
        pl.semaphore_wait(barrier_sem, 2)

        def exchange(idx, partner, src, dst):
            rdma = pltpu.make_async_remote_copy(
                src_ref=src, dst_ref=dst,
                send_sem=send_sems.at[idx], recv_sem=recv_sems.at[idx],
                device_id=(partner,), device_id_type=pl.DeviceIdType.MESH,
            )
            rdma.start()
            return rdma

        L = pl.ds(0, DH)
        R = pl.ds(DH, DH)

        xdma.wait()
        xfull[pl.ds(mychunk, M_PER), :] = xf32[:, :].astype(BF)
        s1a = exchange(0, a, xfull.at[pl.ds(mychunk, M_PER)],
                       xfull.at[pl.ds(mychunk, M_PER)])
        s1b = exchange(1, b, xfull.at[pl.ds(mychunk, M_PER)],
                       xfull.at[pl.ds(mychunk, M_PER)])
        s1a.wait()
        s1b.wait()
        s2a = exchange(2, a, xfull.at[pl.ds(bbase, M_PER), L],
                       xfull.at[pl.ds(bbase, M_PER), L])
        s2b = exchange(3, b, xfull.at[pl.ds(abase, M_PER), R],
                       xfull.at[pl.ds(abase, M_PER), R])
        wdmas[0].wait()
        W1 = w1buf[0, :, :].astype(BF)
        hg = jnp.dot(xfull[pl.ds(gbase, HALF), :], W1,
                     preferred_element_type=jnp.float32)
        hbuf[pl.ds(gbase, HALF), :] = jnp.maximum(hg, 0.0).astype(BF)
        s2a.wait()
        s2b.wait()
        ho = jnp.dot(xfull[pl.ds(obase, HALF), :], W1,
                     preferred_element_type=jnp.float32)
        hbuf[pl.ds(obase, HALF), :] = jnp.maximum(ho, 0.0).astype(BF)

        def halfmm(src, r, w):
            return jnp.dot(src[pl.ds(r * HALF, HALF), :], w,
                           preferred_element_type=jnp.float32)

        def allreduce_fused(lyr, sem0):
            wdmas[2 * lyr + 1].wait()
            W2 = w2buf[lyr, :, :].astype(BF)
            zbuf[pl.ds(0, HALF), :] = halfmm(hbuf, 0, W2).astype(BF)
            e1_r0 = exchange(sem0, b,
                             zbuf.at[pl.ds(0, HALF)],
                             recvbuf.at[0, pl.ds(0, HALF)])
            zbuf[pl.ds(HALF, HALF), :] = halfmm(hbuf, 1, W2).astype(BF)
            e1_r1 = exchange(sem0 + 1, a,
                             zbuf.at[pl.ds(HALF, HALF)],
                             recvbuf.at[0, pl.ds(HALF, HALF)])
            e1_r0.wait()
            zbuf[pl.ds(0, HALF), :] = (
                zbuf[pl.ds(0, HALF), :] + recvbuf[0, pl.ds(0, HALF), :]
            )
            e2_r0 = exchange(sem0 + 2, a,
                             zbuf.at[pl.ds(0, HALF)],
                             recvbuf.at[1, pl.ds(0, HALF)])
            e1_r1.wait()
            zbuf[pl.ds(HALF, HALF), :] = (
                zbuf[pl.ds(HALF, HALF), :] + recvbuf[0, pl.ds(HALF, HALF), :]
            )
            e2_r1 = exchange(sem0 + 3, b,
                             zbuf.at[pl.ds(HALF, HALF)],
                             recvbuf.at[1, pl.ds(HALF, HALF)])
            wdmas[2 * lyr + 2].wait()
            W1n = w1buf[lyr + 1, :, :].astype(BF)
            e2_r0.wait()
            xfull[pl.ds(0, HALF), :] = (
                zbuf[pl.ds(0, HALF), :] + recvbuf[1, pl.ds(0, HALF), :]
            )
            h0 = halfmm(xfull, 0, W1n)
            hbuf[pl.ds(0, HALF), :] = jnp.maximum(h0, 0.0).astype(BF)
            e2_r1.wait()
            xfull[pl.ds(HALF, HALF), :] = (
                zbuf[pl.ds(HALF, HALF), :] + recvbuf[1, pl.ds(HALF, HALF), :]
            )
            h1 = halfmm(xfull, 1, W1n)
            hbuf[pl.ds(HALF, HALF), :] = jnp.maximum(h1, 0.0).astype(BF)

        allreduce_fused(0, 4)
        allreduce_fused(1, 8)

        wdmas[5].wait()
        W2 = w2buf[2, :, :].astype(BF)
        zo = jnp.dot(hbuf[pl.ds(obase, HALF), :], W2,
                     preferred_element_type=jnp.float32)
        zbuf[pl.ds(obase, HALF), :] = zo.astype(BF)
        eL1 = exchange(12, b, zbuf.at[pl.ds(obase, HALF), L],
                       recvbuf.at[0, pl.ds(obase, HALF), L])
        eR1d = exchange(13, a, zbuf.at[pl.ds(dbase, M_PER), R],
                        recvbuf.at[0, pl.ds(dbase, M_PER), R])
        zg = jnp.dot(hbuf[pl.ds(gbase, HALF), :], W2,
                     preferred_element_type=jnp.float32)
        zbuf[pl.ds(gbase, HALF), :] = zg.astype(BF)
        eR1a = exchange(14, a, zbuf.at[pl.ds(abase, M_PER), R],
                        recvbuf.at[0, pl.ds(abase, M_PER), R])
        eL1.wait()
        zbuf[pl.ds(gbase, HALF), L] = (
            zbuf[pl.ds(gbase, HALF), L]
            + recvbuf[0, pl.ds(gbase, HALF), L]
        )
        eL2 = exchange(15, a, zbuf.at[pl.ds(abase, M_PER), L],
                       recvbuf.at[1, pl.ds(abase, M_PER), L])
        eR1d.wait()
        eR1a.wait()
        zbuf[pl.ds(mychunk, M_PER), R] = (
            zbuf[pl.ds(mychunk, M_PER), R]
            + recvbuf[0, pl.ds(mychunk, M_PER), R]
        )
        zbuf[pl.ds(bbase, M_PER), R] = (
            zbuf[pl.ds(bbase, M_PER), R]
            + recvbuf[0, pl.ds(bbase, M_PER), R]
        )
        eR2 = exchange(16, b, zbuf.at[pl.ds(bbase, M_PER), R],
                       recvbuf.at[1, pl.ds(bbase, M_PER), R])
        eL2.wait()
        outstage[:, L] = (
            zbuf[pl.ds(mychunk, M_PER), L]
            + recvbuf[1, pl.ds(mychunk, M_PER), L]
        )
        eR2.wait()
        outstage[:, R] = (
            zbuf[pl.ds(mychunk, M_PER), R]
            + recvbuf[1, pl.ds(mychunk, M_PER), R]
        )
        odma = pltpu.make_async_copy(outstage, out_ref, wsems.at[7])
        odma.start()
        odma.wait()

    hbm = pltpu.MemorySpace.HBM
    args = tuple(
        pltpu.with_memory_space_constraint(v, hbm)
        for v in (x, Win0, Wout0, Win1, Wout1, Win2, Wout2)
    )
    return pl.pallas_call(
        body,
        out_shape=jax.ShapeDtypeStruct((M_PER, D), BF),
        in_specs=[pl.BlockSpec(memory_space=hbm)] * 7,
        out_specs=pl.BlockSpec(memory_space=hbm),
        scratch_shapes=[
            pltpu.VMEM((M, D), BF),
            pltpu.VMEM((M, H_PER), BF),
            pltpu.VMEM((M, D), BF),
            pltpu.VMEM((2, M, D), BF),
            pltpu.VMEM((M_PER, D), jnp.float32),
            pltpu.VMEM((M_PER, D), BF),
            pltpu.VMEM((3, D, H_PER), jnp.float32),
            pltpu.VMEM((3, H_PER, D), jnp.float32),
            pltpu.SemaphoreType.DMA((8,)),
            pltpu.SemaphoreType.DMA((17,)),
            pltpu.SemaphoreType.DMA((17,)),
        ],
        compiler_params=pltpu.CompilerParams(collective_id=0),
    )(*args)


# device time: 27996 ns/iter; 1.5393x vs baseline; 1.0278x over previous
import jax
import jax.numpy as jnp
from jax import lax
from jax.experimental import pallas as pl
from jax.experimental.pallas import tpu as pltpu

N_DEV = 4
M_PER = 64
M = N_DEV * M_PER
HALF = M // 2
D = 512
DH = D // 2
H_PER = 1024
BF = jnp.bfloat16


def kernel(x, Win0, Wout0, Win1, Wout1, Win2, Wout2):
    def body(x_ref, win0_ref, wout0_ref, win1_ref, wout1_ref,
             win2_ref, wout2_ref, out_ref,
             xfull, hbuf, zbuf, recvbuf,
             xf32, outstage, w1buf, w2buf, wsems, send_sems, recv_sems):
        p = lax.axis_index("i")
        a = p ^ 1
        b = 3 - p
        mychunk = p * M_PER
        abase = a * M_PER
        bbase = b * M_PER
        dbase = (p ^ 2) * M_PER
        gbase = (p // 2) * HALF
        obase = (1 - p // 2) * HALF

        xdma = pltpu.make_async_copy(x_ref, xf32, wsems.at[6])
        xdma.start()
        wdmas = []
        for i, (src, dst) in enumerate((
                (win0_ref, w1buf.at[0]), (wout0_ref, w2buf.at[0]),
                (win1_ref, w1buf.at[1]), (wout1_ref, w2buf.at[1]),
                (win2_ref, w1buf.at[2]), (wout2_ref, w2buf.at[2]))):
            dma = pltpu.make_async_copy(src, dst, wsems.at[i])
            dma.start()
            wdmas.append(dma)

        barrier_sem = pltpu.get_barrier_semaphore()
        for nbr in (a, b):
            pl.semaphore_signal(
                barrier_sem, inc=1,
                device_id=(nbr,), device_id_type=pl.DeviceIdType.MESH,
            )
        pl.semaphore_wait(barrier_sem, 2)

        def exchange(idx, partner, src, dst):
            rdma = pltpu.make_async_remote_copy(
                src_ref=src, dst_ref=dst,
                send_sem=send_sems.at[idx], recv_sem=recv_sems.at[idx],
                device_id=(partner,), device_id_type=pl.DeviceIdType.MESH,
            )
            rdma.start()
            return rdma

        L = pl.ds(0, DH)
        R = pl.ds(DH, DH)
        R0 = pl.ds(0, HALF)
        R1 = pl.ds(HALF, HALF)

        def relu_h(rows, n_rows, w):
            h = jnp.dot(xfull[pl.ds(rows, n_rows), :], w,
                        preferred_element_type=jnp.float32)
            hbuf[pl.ds(rows, n_rows), :] = jnp.maximum(h, 0.0).astype(BF)

        xdma.wait()
        xfull[pl.ds(mychunk, M_PER), :] = xf32[:, :].astype(BF)
        s1a = exchange(0, a, xfull.at[pl.ds(mychunk, M_PER)],
                       xfull.at[pl.ds(mychunk, M_PER)])
        s1b = exchange(1, b, xfull.at[pl.ds(mychunk, M_PER)],
                       xfull.at[pl.ds(mychunk, M_PER)])
        s1a.wait()
        s1b.wait()
        s2a = exchange(2, a, xfull.at[pl.ds(bbase, M_PER), L],
                       xfull.at[pl.ds(bbase, M_PER), L])
        s2b = exchange(3, b, xfull.at[pl.ds(abase, M_PER), R],
                       xfull.at[pl.ds(abase, M_PER), R])
        wdmas[0].wait()
        W1 = w1buf[0, :, :].astype(BF)
        relu_h(gbase, HALF, W1)
        relu_h(bbase, M_PER, W1)
        s2a.wait()
        s2b.wait()
        relu_h(dbase, M_PER, W1)

        def halfmm(r, w):
            return jnp.dot(hbuf[pl.ds(r * HALF, HALF), :], w,
                           preferred_element_type=jnp.float32)

        def allreduce_fused(lyr, sem0):
            s1 = lyr % 2
            s2 = 2 + lyr % 2
            wdmas[2 * lyr + 1].wait()
            W2 = w2buf[lyr, :, :].astype(BF)
            zbuf[R0, :] = halfmm(0, W2).astype(BF)
            e1L0 = exchange(sem0 + 0, b, zbuf.at[R0, L],
                            recvbuf.at[s1, R0, L])
            e1R0 = exchange(sem0 + 1, b, zbuf.at[R0, R],
                            recvbuf.at[s1, R0, R])
            zbuf[R1, :] = halfmm(1, W2).astype(BF)
            e1L1 = exchange(sem0 + 4, a, zbuf.at[R1, L],
                            recvbuf.at[s1, R1, L])
            e1R1 = exchange(sem0 + 5, a, zbuf.at[R1, R],
                            recvbuf.at[s1, R1, R])
            e1L0.wait()
            zbuf[R0, L] = zbuf[R0, L] + recvbuf[s1, R0, L]
            e2L0 = exchange(sem0 + 2, a, zbuf.at[R0, L],
                            recvbuf.at[s2, R0, L])
            e1R0.wait()
            zbuf[R0, R] = zbuf[R0, R] + recvbuf[s1, R0, R]
            e2R0 = exchange(sem0 + 3, a, zbuf.at[R0, R],
                            recvbuf.at[s2, R0, R])
            e1L1.wait()
            zbuf[R1, L] = zbuf[R1, L] + recvbuf[s1, R1, L]
            e2L1 = exchange(sem0 + 6, b, zbuf.at[R1, L],
                            recvbuf.at[s2, R1, L])
            e1R1.wait()
            zbuf[R1, R] = zbuf[R1, R] + recvbuf[s1, R1, R]
            e2R1 = exchange(sem0 + 7, b, zbuf.at[R1, R],
                            recvbuf.at[s2, R1, R])
            wdmas[2 * lyr + 2].wait()
            W1n = w1buf[lyr + 1, :, :].astype(BF)
            e2L0.wait()
            xfull[R0, L] = zbuf[R0, L] + recvbuf[s2, R0, L]
            hp0 = jnp.dot(xfull[R0, L], W1n[:DH, :],
                          preferred_element_type=jnp.float32)
            e2R0.wait()
            xfull[R0, R] = zbuf[R0, R] + recvbuf[s2, R0, R]
            hp0 = hp0 + jnp.dot(xfull[R0, R], W1n[DH:, :],
                                preferred_element_type=jnp.float32)
            hbuf[R0, :] = jnp.maximum(hp0, 0.0).astype(BF)
            e2L1.wait()
            xfull[R1, L] = zbuf[R1, L] + recvbuf[s2, R1, L]
            hp1 = jnp.dot(xfull[R1, L], W1n[:DH, :],
                          preferred_element_type=jnp.float32)
            e2R1.wait()
            xfull[R1, R] = zbuf[R1, R] + recvbuf[s2, R1, R]
            hp1 = hp1 + jnp.dot(xfull[R1, R], W1n[DH:, :],
                                preferred_element_type=jnp.float32)
            hbuf[R1, :] = jnp.maximum(hp1, 0.0).astype(BF)

        allreduce_fused(0, 4)
        allreduce_fused(1, 12)

        wdmas[5].wait()
        W2 = w2buf[2, :, :].astype(BF)
        zo = jnp.dot(hbuf[pl.ds(obase, HALF), :], W2,
                     preferred_element_type=jnp.float32)
        zbuf[pl.ds(obase, HALF), :] = zo.astype(BF)
        eL1 = exchange(20, b, zbuf.at[pl.ds(obase, HALF), L],
                       recvbuf.at[0, pl.ds(obase, HALF), L])
        eR1d = exchange(21, a, zbuf.at[pl.ds(dbase, M_PER), R],
                        recvbuf.at[0, pl.ds(dbase, M_PER), R])
        zg = jnp.dot(hbuf[pl.ds(gbase, HALF), :], W2,
                     preferred_element_type=jnp.float32)
        zbuf[pl.ds(gbase, HALF), :] = zg.astype(BF)
        eR1a = exchange(22, a, zbuf.at[pl.ds(abase, M_PER), R],
                        recvbuf.at[0, pl.ds(abase, M_PER), R])
        eL1.wait()
        zbuf[pl.ds(gbase, HALF), L] = (
            zbuf[pl.ds(gbase, HALF), L]
            + recvbuf[0, pl.ds(gbase, HALF), L]
        )
        eL2 = exchange(23, a, zbuf.at[pl.ds(abase, M_PER), L],
                       recvbuf.at[2, pl.ds(abase, M_PER), L])
        eR1d.wait()
        eR1a.wait()
        zbuf[pl.ds(mychunk, M_PER), R] = (
            zbuf[pl.ds(mychunk, M_PER), R]
            + recvbuf[0, pl.ds(mychunk, M_PER), R]
        )
        zbuf[pl.ds(bbase, M_PER), R] = (
            zbuf[pl.ds(bbase, M_PER), R]
            + recvbuf[0, pl.ds(bbase, M_PER), R]
        )
        eR2 = exchange(24, b, zbuf.at[pl.ds(bbase, M_PER), R],
                       recvbuf.at[2, pl.ds(bbase, M_PER), R])
        eL2.wait()
        outstage[:, L] = (
            zbuf[pl.ds(mychunk, M_PER), L]
            + recvbuf[2, pl.ds(mychunk, M_PER), L]
        )
        eR2.wait()
        outstage[:, R] = (
            zbuf[pl.ds(mychunk, M_PER), R]
            + recvbuf[2, pl.ds(mychunk, M_PER), R]
        )
        odma = pltpu.make_async_copy(outstage, out_ref, wsems.at[7])
        odma.start()
        odma.wait()

    hbm = pltpu.MemorySpace.HBM
    args = tuple(
        pltpu.with_memory_space_constraint(v, hbm)
        for v in (x, Win0, Wout0, Win1, Wout1, Win2, Wout2)
    )
    return pl.pallas_call(
        body,
        out_shape=jax.ShapeDtypeStruct((M_PER, D), BF),
        in_specs=[pl.BlockSpec(memory_space=hbm)] * 7,
        out_specs=pl.BlockSpec(memory_space=hbm),
        scratch_shapes=[
            pltpu.VMEM((M, D), BF),
            pltpu.VMEM((M, H_PER), BF),
            pltpu.VMEM((M, D), BF),
            pltpu.VMEM((4, M, D), BF),
            pltpu.VMEM((M_PER, D), jnp.float32),
            pltpu.VMEM((M_PER, D), BF),
            pltpu.VMEM((3, D, H_PER), jnp.float32),
            pltpu.VMEM((3, H_PER, D), jnp.float32),
            pltpu.SemaphoreType.DMA((8,)),
            pltpu.SemaphoreType.DMA((25,)),
            pltpu.SemaphoreType.DMA((25,)),
        ],
        compiler_params=pltpu.CompilerParams(collective_id=0),
    )(*args)


# device time: 27961 ns/iter; 1.5412x vs baseline; 1.0013x over previous
import jax
import jax.numpy as jnp
from jax import lax
from jax.experimental import pallas as pl
from jax.experimental.pallas import tpu as pltpu

N_DEV = 4
M_PER = 64
M = N_DEV * M_PER
HALF = M // 2
D = 512
DH = D // 2
H_PER = 1024
BF = jnp.bfloat16


def kernel(x, Win0, Wout0, Win1, Wout1, Win2, Wout2):
    def body(x_ref, win0_ref, wout0_ref, win1_ref, wout1_ref,
             win2_ref, wout2_ref, out_ref,
             xfull, hbuf, zbuf, recvbuf,
             xf32, outstage, w1buf, w2buf, wsems, send_sems, recv_sems):
        p = lax.axis_index("i")
        a = p ^ 1
        b = 3 - p
        mychunk = p * M_PER
        abase = a * M_PER
        bbase = b * M_PER
        dbase = (p ^ 2) * M_PER
        gbase = (p // 2) * HALF
        obase = (1 - p // 2) * HALF

        xdma = pltpu.make_async_copy(x_ref, xf32, wsems.at[6])
        xdma.start()
        wdmas = []
        for i, (src, dst) in enumerate((
                (win0_ref, w1buf.at[0]), (wout0_ref, w2buf.at[0]),
                (win1_ref, w1buf.at[1]), (wout1_ref, w2buf.at[1]),
                (win2_ref, w1buf.at[2]), (wout2_ref, w2buf.at[2]))):
            dma = pltpu.make_async_copy(src, dst, wsems.at[i])
            dma.start()
            wdmas.append(dma)

        barrier_sem = pltpu.get_barrier_semaphore()
        for nbr in (a, b):
            pl.semaphore_signal(
                barrier_sem, inc=1,
                device_id=(nbr,), device_id_type=pl.DeviceIdType.MESH,
            )

        def exchange(idx, partner, src, dst):
            rdma = pltpu.make_async_remote_copy(
                src_ref=src, dst_ref=dst,
                send_sem=send_sems.at[idx], recv_sem=recv_sems.at[idx],
                device_id=(partner,), device_id_type=pl.DeviceIdType.MESH,
            )
            rdma.start()
            return rdma

        L = pl.ds(0, DH)
        R = pl.ds(DH, DH)
        R0 = pl.ds(0, HALF)
        R1 = pl.ds(HALF, HALF)

        def relu_h(rows, n_rows, w):
            h = jnp.dot(xfull[pl.ds(rows, n_rows), :], w,
                        preferred_element_type=jnp.float32)
            hbuf[pl.ds(rows, n_rows), :] = jnp.maximum(h, 0.0).astype(BF)

        xdma.wait()
        xfull[pl.ds(mychunk, M_PER), :] = xf32[:, :].astype(BF)
        pl.semaphore_wait(barrier_sem, 2)
        s1a = exchange(0, a, xfull.at[pl.ds(mychunk, M_PER)],
                       xfull.at[pl.ds(mychunk, M_PER)])
        s1b = exchange(1, b, xfull.at[pl.ds(mychunk, M_PER)],
                       xfull.at[pl.ds(mychunk, M_PER)])
        s1a.wait()
        s1b.wait()
        s2a = exchange(2, a, xfull.at[pl.ds(bbase, M_PER), L],
                       xfull.at[pl.ds(bbase, M_PER), L])
        s2b = exchange(3, b, xfull.at[pl.ds(abase, M_PER), R],
                       xfull.at[pl.ds(abase, M_PER), R])
        wdmas[0].wait()
        W1 = w1buf[0, :, :].astype(BF)
        relu_h(gbase, HALF, W1)
        relu_h(bbase, M_PER, W1)
        s2a.wait()
        s2b.wait()
        relu_h(dbase, M_PER, W1)

        def halfmm(r, w):
            return jnp.dot(hbuf[pl.ds(r * HALF, HALF), :], w,
                           preferred_element_type=jnp.float32)

        def allreduce_fused(lyr, sem0):
            s1 = lyr % 2
            s2 = 2 + lyr % 2
            wdmas[2 * lyr + 1].wait()
            W2 = w2buf[lyr, :, :].astype(BF)
            zbuf[R0, :] = halfmm(0, W2).astype(BF)
            e1L0 = exchange(sem0 + 0, b, zbuf.at[R0, L],
                            recvbuf.at[s1, R0, L])
            e1R0 = exchange(sem0 + 1, b, zbuf.at[R0, R],
                            recvbuf.at[s1, R0, R])
            zbuf[R1, :] = halfmm(1, W2).astype(BF)
            e1L1 = exchange(sem0 + 4, a, zbuf.at[R1, L],
                            recvbuf.at[s1, R1, L])
            e1R1 = exchange(sem0 + 5, a, zbuf.at[R1, R],
                            recvbuf.at[s1, R1, R])
            e1L0.wait()
            zbuf[R0, L] = zbuf[R0, L] + recvbuf[s1, R0, L]
            e2L0 = exchange(sem0 + 2, a, zbuf.at[R0, L],
                            recvbuf.at[s2, R0, L])
            e1R0.wait()
            zbuf[R0, R] = zbuf[R0, R] + recvbuf[s1, R0, R]
            e2R0 = exchange(sem0 + 3, a, zbuf.at[R0, R],
                            recvbuf.at[s2, R0, R])
            e1L1.wait()
            zbuf[R1, L] = zbuf[R1, L] + recvbuf[s1, R1, L]
            e2L1 = exchange(sem0 + 6, b, zbuf.at[R1, L],
                            recvbuf.at[s2, R1, L])
            e1R1.wait()
            zbuf[R1, R] = zbuf[R1, R] + recvbuf[s1, R1, R]
            e2R1 = exchange(sem0 + 7, b, zbuf.at[R1, R],
                            recvbuf.at[s2, R1, R])
            wdmas[2 * lyr + 2].wait()
            W1n = w1buf[lyr + 1, :, :].astype(BF)
            e2L0.wait()
            xfull[R0, L] = zbuf[R0, L] + recvbuf[s2, R0, L]
            hp0 = jnp.dot(xfull[R0, L], W1n[:DH, :],
                          preferred_element_type=jnp.float32)
            e2R0.wait()
            xfull[R0, R] = zbuf[R0, R] + recvbuf[s2, R0, R]
            hp0 = hp0 + jnp.dot(xfull[R0, R], W1n[DH:, :],
                                preferred_element_type=jnp.float32)
            hbuf[R0, :] = jnp.maximum(hp0, 0.0).astype(BF)
            e2L1.wait()
            xfull[R1, L] = zbuf[R1, L] + recvbuf[s2, R1, L]
            hp1 = jnp.dot(xfull[R1, L], W1n[:DH, :],
                          preferred_element_type=jnp.float32)
            e2R1.wait()
            xfull[R1, R] = zbuf[R1, R] + recvbuf[s2, R1, R]
            hp1 = hp1 + jnp.dot(xfull[R1, R], W1n[DH:, :],
                                preferred_element_type=jnp.float32)
            hbuf[R1, :] = jnp.maximum(hp1, 0.0).astype(BF)

        allreduce_fused(0, 4)
        allreduce_fused(1, 12)

        wdmas[5].wait()
        W2 = w2buf[2, :, :].astype(BF)
        ho_ = hbuf[pl.ds(obase, HALF), :]
        zoL = jnp.dot(ho_, W2[:, :DH], preferred_element_type=jnp.float32)
        zbuf[pl.ds(obase, HALF), L] = zoL.astype(BF)
        eL1 = exchange(20, b, zbuf.at[pl.ds(obase, HALF), L],
                       recvbuf.at[0, pl.ds(obase, HALF), L])
        zoR = jnp.dot(ho_, W2[:, DH:], preferred_element_type=jnp.float32)
        zbuf[pl.ds(obase, HALF), R] = zoR.astype(BF)
        eR1d = exchange(21, a, zbuf.at[pl.ds(dbase, M_PER), R],
                        recvbuf.at[0, pl.ds(dbase, M_PER), R])
        zg = jnp.dot(hbuf[pl.ds(gbase, HALF), :], W2,
                     preferred_element_type=jnp.float32)
        zbuf[pl.ds(gbase, HALF), :] = zg.astype(BF)
        eR1a = exchange(22, a, zbuf.at[pl.ds(abase, M_PER), R],
                        recvbuf.at[0, pl.ds(abase, M_PER), R])
        eL1.wait()
        zbuf[pl.ds(gbase, HALF), L] = (
            zbuf[pl.ds(gbase, HALF), L]
            + recvbuf[0, pl.ds(gbase, HALF), L]
        )
        eL2 = exchange(23, a, zbuf.at[pl.ds(abase, M_PER), L],
                       recvbuf.at[2, pl.ds(abase, M_PER), L])
        eR1d.wait()
        eR1a.wait()
        zbuf[pl.ds(mychunk, M_PER), R] = (
            zbuf[pl.ds(mychunk, M_PER), R]
            + recvbuf[0, pl.ds(mychunk, M_PER), R]
        )
        zbuf[pl.ds(bbase, M_PER), R] = (
            zbuf[pl.ds(bbase, M_PER), R]
            + recvbuf[0, pl.ds(bbase, M_PER), R]
        )
        eR2 = exchange(24, b, zbuf.at[pl.ds(bbase, M_PER), R],
                       recvbuf.at[2, pl.ds(bbase, M_PER), R])
        eL2.wait()
        outstage[:, L] = (
            zbuf[pl.ds(mychunk, M_PER), L]
            + recvbuf[2, pl.ds(mychunk, M_PER), L]
        )
        eR2.wait()
        outstage[:, R] = (
            zbuf[pl.ds(mychunk, M_PER), R]
            + recvbuf[2, pl.ds(mychunk, M_PER), R]
        )
        odma = pltpu.make_async_copy(outstage, out_ref, wsems.at[7])
        odma.start()
        odma.wait()

    hbm = pltpu.MemorySpace.HBM
    args = tuple(
        pltpu.with_memory_space_constraint(v, hbm)
        for v in (x, Win0, Wout0, Win1, Wout1, Win2, Wout2)
    )
    return pl.pallas_call(
        body,
        out_shape=jax.ShapeDtypeStruct((M_PER, D), BF),
        in_specs=[pl.BlockSpec(memory_space=hbm)] * 7,
        out_specs=pl.BlockSpec(memory_space=hbm),
        scratch_shapes=[
            pltpu.VMEM((M, D), BF),
            pltpu.VMEM((M, H_PER), BF),
            pltpu.VMEM((M, D), BF),
            pltpu.VMEM((4, M, D), BF),
            pltpu.VMEM((M_PER, D), jnp.float32),
            pltpu.VMEM((M_PER, D), BF),
            pltpu.VMEM((3, D, H_PER), jnp.float32),
            pltpu.VMEM((3, H_PER, D), jnp.float32),
            pltpu.SemaphoreType.DMA((8,)),
            pltpu.SemaphoreType.DMA((25,)),
            pltpu.SemaphoreType.DMA((25,)),
        ],
        compiler_params=pltpu.CompilerParams(collective_id=0),
    )(*args)
